# baseline (speedup 1.0000x reference)
import jax
import jax.numpy as jnp
from jax.experimental import pallas as pl
from jax.experimental.pallas import tpu as pltpu
from jax.experimental.pallas import tpu_sc as plsc

N_INPUTS = 784
N_HIDDEN = 2000
N_OUTPUTS = 10
N_NEURONS = N_HIDDEN + N_OUTPUTS
T_MAX = 12
TAU = 10.0
DT = 1.0
THRESH = 0.5
VMAX = 1.0
D_MIN = 3
N_DELAYS = 5
B = 64
S_PAD = 2048
K_PAD = 896


M_TOTAL = N_DELAYS * S_PAD * S_PAD
NC = 2
NS = 16
LANE = 128
ZCH = 16384
M_HALF = M_TOTAL // NC
M_WORKER = M_TOTAL // (NC * NS)
ZREP = M_WORKER // ZCH


def _make_wd_scatter(n_chunks):
    mesh = plsc.VectorSubcoreMesh(core_axis_name="c", subcore_axis_name="s")

    def body(idx_hbm, val_hbm, out_hbm, zbuf, idx_v, val_v, zsem, ssem):
        c = jax.lax.axis_index("c")
        s = jax.lax.axis_index("s")
        base = (c * NS + s) * M_WORKER

        @pl.loop(0, ZCH // 16)
        def _fill(i):
            zbuf[pl.ds(i * 16, 16)] = jnp.zeros((16,), jnp.float32)

        pltpu.sync_copy(idx_hbm.at[c, s], idx_v)
        pltpu.sync_copy(val_hbm.at[c, s], val_v)

        zh = [pltpu.async_copy(zbuf, out_hbm.at[pl.ds(base + r * ZCH, ZCH)],
                               zsem) for r in range(ZREP)]
        for h in zh:
            h.wait()
        plsc.subcore_barrier()

        sh = [pltpu.async_copy(val_v.at[j], out_hbm.at[idx_v.at[j]], ssem)
              for j in range(n_chunks)]
        for h in sh:
            h.wait()

    return pl.kernel(
        body,
        out_type=jax.ShapeDtypeStruct((M_TOTAL,), jnp.float32),
        mesh=mesh,
        scratch_types=[
            pltpu.VMEM((ZCH,), jnp.float32),
            pltpu.VMEM((n_chunks, LANE), jnp.int32),
            pltpu.VMEM((n_chunks, LANE), jnp.float32),
            pltpu.SemaphoreType.DMA,
            pltpu.SemaphoreType.DMA,
        ],
    )


def _build_wd(flat_idx, w_eff):
    e_total = flat_idx.shape[0]
    n_chunks = -(-e_total // (NS * LANE))
    cap = NS * n_chunks * LANE

    fi = jnp.full((cap,), -1, jnp.int32).at[:e_total].set(flat_idx)
    wv = jnp.zeros((cap,), jnp.float32).at[:e_total].set(w_eff)
    slot = jnp.arange(cap, dtype=jnp.int32)
    slot_s = slot // (n_chunks * LANE)
    slot_j = (slot // LANE) % n_chunks
    slot_l = slot % LANE

    is_edge = fi >= 0
    sent = ((slot_s * 64 + slot_j * LANE + slot_l) % (M_TOTAL // S_PAD)
            ) * S_PAD + N_NEURONS + slot_l % (S_PAD - N_NEURONS)
    idx_one = jnp.where(is_edge, fi, sent).reshape(NS, n_chunks, LANE)
    val_one = jnp.where(is_edge, wv, 0.0).reshape(NS, n_chunks, LANE)
    idx_h = jnp.stack([idx_one, idx_one])
    val_h = jnp.stack([val_one, val_one])
    wd_flat = _make_wd_scatter(n_chunks)(idx_h, val_h)
    return wd_flat.reshape(N_DELAYS, S_PAD, S_PAD)


H_SPLIT = 1024
N_UNITS = N_DELAYS * 2
N_BUFS = 4


def _snn_kernel(icur_ref, wd_hbm, out_ref,
                S_ref, Vv_ref, Vm_ref, Iacc_ref, wbufs, sems):
    t = pl.program_id(0)
    @pl.when(t == 0)
    def _init():
        S_ref[...] = jnp.zeros_like(S_ref)
        Vv_ref[...] = jnp.zeros_like(Vv_ref)
        Vm_ref[...] = jnp.zeros_like(Vm_ref)
        out_ref[...] = jnp.zeros_like(out_ref)

    Iacc_ref[...] = jnp.zeros_like(Iacc_ref)

    @pl.when(t >= D_MIN + 2)
    def _deliver():
        acts = [jnp.any(S_ref[k] == jnp.float32(D_MIN + k))
                for k in range(N_DELAYS)]

        def _copy(u):
            k, h = u // 2, u % 2
            return pltpu.make_async_copy(
                wd_hbm.at[k, pl.ds(h * H_SPLIT, H_SPLIT), :],
                wbufs.at[u % N_BUFS],
                sems.at[u % N_BUFS])

        for u in range(N_BUFS):
            @pl.when(acts[u // 2])
            def _start(u=u):
                _copy(u).start()

        for u in range(N_UNITS):
            k, h = u // 2, u % 2

            @pl.when(acts[k])
            def _unit(u=u, k=k, h=h):
                _copy(u).wait()
                arrk = (S_ref[k] == jnp.float32(D_MIN + k)).astype(jnp.float32)
                deliver = Vv_ref[k] * arrk
                dsl = deliver[:, h * H_SPLIT:(h + 1) * H_SPLIT]
                Iacc_ref[...] += jax.lax.dot_general(
                    dsl, wbufs[u % N_BUFS], (((1,), (0,)), ((), ())),
                    precision=jax.lax.Precision.HIGHEST,
                    preferred_element_type=jnp.float32)

            if u + N_BUFS < N_UNITS:
                @pl.when(acts[(u + N_BUFS) // 2])
                def _next(u=u):
                    _copy(u + N_BUFS).start()

    def _finish_step():
        I_syn = Iacc_ref[...]
        inject = (t % 3) == 2
        I_syn = I_syn + jnp.where(inject, icur_ref[...], 0.0)
        Vm = Vm_ref[...]
        Vm = Vm + (-Vm + I_syn) * (DT / TAU)
        V_exc = jnp.maximum(0.0, Vm - THRESH)
        col = jax.lax.broadcasted_iota(jnp.int32, (B, S_PAD), 1)
        fired = (V_exc > 0.0) & (col < N_HIDDEN)

        S = S_ref[...]
        V = Vv_ref[...]
        dvals = (jax.lax.broadcasted_iota(
            jnp.int32, (N_DELAYS, B, S_PAD), 0) + D_MIN).astype(jnp.float32)
        arrived = S == dvals
        idle = S == 0.0
        newS = fired[None] & idle
        live = (~arrived).astype(jnp.float32)
        S = S * live
        V = V * live

        out_mask = ((col >= N_HIDDEN) & (col < N_NEURONS)).astype(jnp.float32)
        out_ref[...] += Vm * out_mask

        firedf = fired.astype(jnp.float32)
        Vm = Vm - (Vm * firedf + 0.2 * firedf)
        newSf = newS.astype(jnp.float32)
        S = S + (S > 0.0).astype(jnp.float32) * (DT * VMAX) + newSf * (DT * VMAX)
        V = V + newSf * V_exc[None]

        S_ref[...] = S
        Vv_ref[...] = V
        Vm_ref[...] = Vm

        @pl.when(t == T_MAX - 1)
        def _done():
            out_ref[...] = out_ref[...] / jnp.float32(T_MAX)

    _finish_step()


def kernel(x, W_e, input_W, L_e, src, tgt, key):
    del key
    d_round = jnp.round(L_e)
    is_int = jnp.abs(L_e - d_round) < 0.25
    d_idx = jnp.clip(d_round.astype(jnp.int32) - D_MIN, 0, N_DELAYS - 1)
    w_eff = jnp.where(is_int, W_e, 0.0)
    flat_idx = (d_idx * S_PAD + src) * S_PAD + tgt
    Wd = _build_wd(flat_idx, w_eff)

    input_currents = x.reshape(B, -1) @ input_W
    icur = jnp.pad(input_currents, ((0, 0), (0, S_PAD - N_HIDDEN)))

    out = pl.pallas_call(
        _snn_kernel,
        grid=(T_MAX,),
        in_specs=[
            pl.BlockSpec((B, S_PAD), lambda t: (0, 0)),
            pl.BlockSpec(memory_space=pl.ANY),
        ],
        out_specs=pl.BlockSpec((B, S_PAD), lambda t: (0, 0)),
        out_shape=jax.ShapeDtypeStruct((B, S_PAD), jnp.float32),
        scratch_shapes=[
            pltpu.VMEM((N_DELAYS, B, S_PAD), jnp.float32),
            pltpu.VMEM((N_DELAYS, B, S_PAD), jnp.float32),
            pltpu.VMEM((B, S_PAD), jnp.float32),
            pltpu.VMEM((B, S_PAD), jnp.float32),
            pltpu.VMEM((N_BUFS, H_SPLIT, S_PAD), jnp.float32),
            pltpu.SemaphoreType.DMA((N_BUFS,)),
        ],
        compiler_params=pltpu.CompilerParams(
            dimension_semantics=("arbitrary",),
            vmem_limit_bytes=100 * 1024 * 1024,
        ),
    )(icur, Wd)
    return out[:, N_HIDDEN:N_NEURONS]


# speedup vs baseline: 1.0944x; 1.0944x over previous
import jax
import jax.numpy as jnp
from jax.experimental import pallas as pl
from jax.experimental.pallas import tpu as pltpu
from jax.experimental.pallas import tpu_sc as plsc

N_INPUTS = 784
N_HIDDEN = 2000
N_OUTPUTS = 10
N_NEURONS = N_HIDDEN + N_OUTPUTS
T_MAX = 12
TAU = 10.0
DT = 1.0
THRESH = 0.5
VMAX = 1.0
D_MIN = 3
N_DELAYS = 5
B = 64
S_PAD = 2048
K_PAD = 896


M_TOTAL = N_DELAYS * S_PAD * S_PAD
NC = 2
NS = 16
LANE = 128
ZCH = 16384
M_HALF = M_TOTAL // NC
M_WORKER = M_TOTAL // (NC * NS)
ZREP = M_WORKER // ZCH


def _make_wd_scatter(n_chunks):
    mesh = plsc.VectorSubcoreMesh(core_axis_name="c", subcore_axis_name="s")

    def body(idx_hbm, val_hbm, out_hbm, zbuf, idx_v, val_v, zsem, ssem):
        c = jax.lax.axis_index("c")
        s = jax.lax.axis_index("s")
        base = (c * NS + s) * M_WORKER

        @pl.loop(0, ZCH // 16)
        def _fill(i):
            zbuf[pl.ds(i * 16, 16)] = jnp.zeros((16,), jnp.float32)

        pltpu.sync_copy(idx_hbm.at[c, s], idx_v)
        pltpu.sync_copy(val_hbm.at[c, s], val_v)

        zh = [pltpu.async_copy(zbuf, out_hbm.at[pl.ds(base + r * ZCH, ZCH)],
                               zsem) for r in range(ZREP)]
        for h in zh:
            h.wait()
        plsc.subcore_barrier()

        sh = [pltpu.async_copy(val_v.at[j], out_hbm.at[idx_v.at[j]], ssem)
              for j in range(n_chunks)]
        for h in sh:
            h.wait()

    return pl.kernel(
        body,
        out_type=jax.ShapeDtypeStruct((M_TOTAL,), jnp.float32),
        mesh=mesh,
        scratch_types=[
            pltpu.VMEM((ZCH,), jnp.float32),
            pltpu.VMEM((n_chunks, LANE), jnp.int32),
            pltpu.VMEM((n_chunks, LANE), jnp.float32),
            pltpu.SemaphoreType.DMA,
            pltpu.SemaphoreType.DMA,
        ],
    )


def _build_wd(flat_idx, w_eff):
    e_total = flat_idx.shape[0]
    n_chunks = -(-e_total // (NS * LANE))
    cap = NS * n_chunks * LANE

    fi = jnp.full((cap,), -1, jnp.int32).at[:e_total].set(flat_idx)
    wv = jnp.zeros((cap,), jnp.float32).at[:e_total].set(w_eff)
    slot = jnp.arange(cap, dtype=jnp.int32)
    slot_s = slot // (n_chunks * LANE)
    slot_j = (slot // LANE) % n_chunks
    slot_l = slot % LANE

    is_edge = fi >= 0
    sent = ((slot_s * 64 + slot_j * LANE + slot_l) % (M_TOTAL // S_PAD)
            ) * S_PAD + N_NEURONS + slot_l % (S_PAD - N_NEURONS)
    idx_one = jnp.where(is_edge, fi, sent).reshape(NS, n_chunks, LANE)
    val_one = jnp.where(is_edge, wv, 0.0).reshape(NS, n_chunks, LANE)
    idx_h = jnp.stack([idx_one, idx_one])
    val_h = jnp.stack([val_one, val_one])
    wd_flat = _make_wd_scatter(n_chunks)(idx_h, val_h)
    return wd_flat.reshape(N_DELAYS, S_PAD, S_PAD)


def _snn_kernel(icur_ref, wd_ref, out_ref,
                S_ref, Vv_ref, Vm_ref, Iacc_ref):
    t = pl.program_id(0)
    k = pl.program_id(1)

    @pl.when((t == 0) & (k == 0))
    def _init():
        S_ref[...] = jnp.zeros_like(S_ref)
        Vv_ref[...] = jnp.zeros_like(Vv_ref)
        Vm_ref[...] = jnp.zeros_like(Vm_ref)
        out_ref[...] = jnp.zeros_like(out_ref)

    @pl.when(k == 0)
    def _zero_acc():
        Iacc_ref[...] = jnp.zeros_like(Iacc_ref)

    @pl.when(t >= D_MIN + 2)
    def _deliver():
        d_val = (D_MIN + k).astype(jnp.float32)
        Sk = S_ref[k]
        arr = Sk == d_val
        @pl.when(jnp.any(arr))
        def _matmul():
            deliver = Vv_ref[k] * arr.astype(jnp.float32)
            Iacc_ref[...] += jax.lax.dot_general(
                deliver, wd_ref[0], (((1,), (0,)), ((), ())),
                precision=jax.lax.Precision.HIGHEST,
                preferred_element_type=jnp.float32)

    @pl.when(k == N_DELAYS - 1)
    def _finish_step():
        I_syn = Iacc_ref[...]
        inject = (t % 3) == 2
        I_syn = I_syn + jnp.where(inject, icur_ref[...], 0.0)
        Vm = Vm_ref[...]
        Vm = Vm + (-Vm + I_syn) * (DT / TAU)
        V_exc = jnp.maximum(0.0, Vm - THRESH)
        col = jax.lax.broadcasted_iota(jnp.int32, (B, S_PAD), 1)
        fired = (V_exc > 0.0) & (col < N_HIDDEN)

        S = S_ref[...]
        V = Vv_ref[...]
        dvals = (jax.lax.broadcasted_iota(
            jnp.int32, (N_DELAYS, B, S_PAD), 0) + D_MIN).astype(jnp.float32)
        arrived = S == dvals
        idle = S == 0.0
        newS = fired[None] & idle
        live = (~arrived).astype(jnp.float32)
        S = S * live
        V = V * live

        out_mask = ((col >= N_HIDDEN) & (col < N_NEURONS)).astype(jnp.float32)
        out_ref[...] += Vm * out_mask

        firedf = fired.astype(jnp.float32)
        Vm = Vm - (Vm * firedf + 0.2 * firedf)
        newSf = newS.astype(jnp.float32)
        S = S + (S > 0.0).astype(jnp.float32) * (DT * VMAX) + newSf * (DT * VMAX)
        V = V + newSf * V_exc[None]

        S_ref[...] = S
        Vv_ref[...] = V
        Vm_ref[...] = Vm

        @pl.when(t == T_MAX - 1)
        def _done():
            out_ref[...] = out_ref[...] / jnp.float32(T_MAX)


def kernel(x, W_e, input_W, L_e, src, tgt, key):
    del key
    d_round = jnp.round(L_e)
    is_int = jnp.abs(L_e - d_round) < 0.25
    d_idx = jnp.clip(d_round.astype(jnp.int32) - D_MIN, 0, N_DELAYS - 1)
    w_eff = jnp.where(is_int, W_e, 0.0)
    flat_idx = (d_idx * S_PAD + src) * S_PAD + tgt
    Wd = _build_wd(flat_idx, w_eff)

    input_currents = x.reshape(B, -1) @ input_W
    icur = jnp.pad(input_currents, ((0, 0), (0, S_PAD - N_HIDDEN)))

    out = pl.pallas_call(
        _snn_kernel,
        grid=(T_MAX, N_DELAYS),
        in_specs=[
            pl.BlockSpec((B, S_PAD), lambda t, k: (0, 0)),
            pl.BlockSpec((1, S_PAD, S_PAD),
                         lambda t, k: (jnp.where(t >= D_MIN + 2, k, 0), 0, 0)),
        ],
        out_specs=pl.BlockSpec((B, S_PAD), lambda t, k: (0, 0)),
        out_shape=jax.ShapeDtypeStruct((B, S_PAD), jnp.float32),
        scratch_shapes=[
            pltpu.VMEM((N_DELAYS, B, S_PAD), jnp.float32),
            pltpu.VMEM((N_DELAYS, B, S_PAD), jnp.float32),
            pltpu.VMEM((B, S_PAD), jnp.float32),
            pltpu.VMEM((B, S_PAD), jnp.float32),
        ],
        compiler_params=pltpu.CompilerParams(
            dimension_semantics=("arbitrary", "arbitrary"),
            vmem_limit_bytes=100 * 1024 * 1024,
        ),
    )(icur, Wd)
    return out[:, N_HIDDEN:N_NEURONS]


# speedup vs baseline: 1.1045x; 1.0093x over previous
import jax
import jax.numpy as jnp
from jax.experimental import pallas as pl
from jax.experimental.pallas import tpu as pltpu
from jax.experimental.pallas import tpu_sc as plsc

N_INPUTS = 784
N_HIDDEN = 2000
N_OUTPUTS = 10
N_NEURONS = N_HIDDEN + N_OUTPUTS
T_MAX = 12
TAU = 10.0
DT = 1.0
THRESH = 0.5
VMAX = 1.0
D_MIN = 3
N_DELAYS = 5
B = 64
S_PAD = 2048
K_PAD = 896


M_TOTAL = N_DELAYS * S_PAD * S_PAD
NC = 2
NS = 16
LANE = 128
ZCH = 16384
M_HALF = M_TOTAL // NC
M_WORKER = M_TOTAL // (NC * NS)
ZREP = M_WORKER // ZCH


def _make_wd_scatter(n_chunks):
    mesh = plsc.VectorSubcoreMesh(core_axis_name="c", subcore_axis_name="s")

    def body(idx_hbm, val_hbm, out_hbm, zbuf, idx_v, val_v, zsem, ssem):
        c = jax.lax.axis_index("c")
        s = jax.lax.axis_index("s")
        base = (c * NS + s) * M_WORKER

        @pl.loop(0, ZCH // 16)
        def _fill(i):
            zbuf[pl.ds(i * 16, 16)] = jnp.zeros((16,), jnp.float32)

        pltpu.sync_copy(idx_hbm.at[c, s], idx_v)
        pltpu.sync_copy(val_hbm.at[c, s], val_v)

        zh = [pltpu.async_copy(zbuf, out_hbm.at[pl.ds(base + r * ZCH, ZCH)],
                               zsem) for r in range(ZREP)]
        for h in zh:
            h.wait()
        plsc.subcore_barrier()

        sh = [pltpu.async_copy(val_v.at[j], out_hbm.at[idx_v.at[j]], ssem)
              for j in range(n_chunks)]
        for h in sh:
            h.wait()

    return pl.kernel(
        body,
        out_type=jax.ShapeDtypeStruct((M_TOTAL,), jnp.float32),
        mesh=mesh,
        scratch_types=[
            pltpu.VMEM((ZCH,), jnp.float32),
            pltpu.VMEM((n_chunks, LANE), jnp.int32),
            pltpu.VMEM((n_chunks, LANE), jnp.float32),
            pltpu.SemaphoreType.DMA,
            pltpu.SemaphoreType.DMA,
        ],
    )


def _build_wd(flat_idx, w_eff):
    e_total = flat_idx.shape[0]
    n_chunks = -(-e_total // (NS * LANE))
    cap = NS * n_chunks * LANE

    fi = jnp.full((cap,), -1, jnp.int32).at[:e_total].set(flat_idx)
    wv = jnp.zeros((cap,), jnp.float32).at[:e_total].set(w_eff)
    slot = jnp.arange(cap, dtype=jnp.int32)
    slot_s = slot // (n_chunks * LANE)
    slot_j = (slot // LANE) % n_chunks
    slot_l = slot % LANE

    is_edge = fi >= 0
    sent = ((slot_s * 64 + slot_j * LANE + slot_l) % (M_TOTAL // S_PAD)
            ) * S_PAD + N_NEURONS + slot_l % (S_PAD - N_NEURONS)
    idx_one = jnp.where(is_edge, fi, sent).reshape(NS, n_chunks, LANE)
    val_one = jnp.where(is_edge, wv, 0.0).reshape(NS, n_chunks, LANE)
    idx_h = jnp.stack([idx_one, idx_one])
    val_h = jnp.stack([val_one, val_one])
    wd_flat = _make_wd_scatter(n_chunks)(idx_h, val_h)
    return wd_flat.reshape(N_DELAYS, S_PAD, S_PAD)


T_WARM = D_MIN + 2


def _warm_kernel(icur_ref, S_out, V_out, Vm_out, acc_out):
    def step(t, carry):
        S, V, Vm, acc = carry
        inject = (t % 3) == 2
        I_syn = jnp.where(inject, icur_ref[...], 0.0)
        Vm = Vm + (-Vm + I_syn) * (DT / TAU)
        V_exc = jnp.maximum(0.0, Vm - THRESH)
        col = jax.lax.broadcasted_iota(jnp.int32, (B, S_PAD), 1)
        fired = (V_exc > 0.0) & (col < N_HIDDEN)
        dvals = (jax.lax.broadcasted_iota(
            jnp.int32, (N_DELAYS, B, S_PAD), 0) + D_MIN).astype(jnp.float32)
        arrived = S == dvals
        idle = S == 0.0
        newS = fired[None] & idle
        live = (~arrived).astype(jnp.float32)
        S = S * live
        V = V * live
        out_mask = ((col >= N_HIDDEN) & (col < N_NEURONS)).astype(jnp.float32)
        acc = acc + Vm * out_mask
        firedf = fired.astype(jnp.float32)
        Vm = Vm - (Vm * firedf + 0.2 * firedf)
        newSf = newS.astype(jnp.float32)
        S = S + (S > 0.0).astype(jnp.float32) * (DT * VMAX) + newSf * (DT * VMAX)
        V = V + newSf * V_exc[None]
        return S, V, Vm, acc

    z3 = jnp.zeros((N_DELAYS, B, S_PAD), jnp.float32)
    z2 = jnp.zeros((B, S_PAD), jnp.float32)
    S, V, Vm, acc = jax.lax.fori_loop(0, T_WARM, step, (z3, z3, z2, z2))
    S_out[...] = S
    V_out[...] = V
    Vm_out[...] = Vm
    acc_out[...] = acc


def _snn_kernel(icur_ref, wd_ref, S_in, V_in, Vm_in, acc_in, out_ref,
                S_ref, Vv_ref, Vm_ref, Iacc_ref):
    t = pl.program_id(0) + T_WARM
    k = pl.program_id(1)

    @pl.when((t == T_WARM) & (k == 0))
    def _init():
        S_ref[...] = S_in[...]
        Vv_ref[...] = V_in[...]
        Vm_ref[...] = Vm_in[...]
        out_ref[...] = acc_in[...]

    @pl.when(k == 0)
    def _zero_acc():
        Iacc_ref[...] = jnp.zeros_like(Iacc_ref)

    d_val = (D_MIN + k).astype(jnp.float32)
    Sk = S_ref[k]
    arr = Sk == d_val
    @pl.when(jnp.any(arr))
    def _matmul():
        deliver = Vv_ref[k] * arr.astype(jnp.float32)
        Iacc_ref[...] += jax.lax.dot_general(
            deliver, wd_ref[0], (((1,), (0,)), ((), ())),
            precision=jax.lax.Precision.HIGHEST,
            preferred_element_type=jnp.float32)

    @pl.when(k == N_DELAYS - 1)
    def _finish_step():
        I_syn = Iacc_ref[...]
        inject = (t % 3) == 2
        I_syn = I_syn + jnp.where(inject, icur_ref[...], 0.0)
        Vm = Vm_ref[...]
        Vm = Vm + (-Vm + I_syn) * (DT / TAU)
        V_exc = jnp.maximum(0.0, Vm - THRESH)
        col = jax.lax.broadcasted_iota(jnp.int32, (B, S_PAD), 1)
        fired = (V_exc > 0.0) & (col < N_HIDDEN)

        S = S_ref[...]
        V = Vv_ref[...]
        dvals = (jax.lax.broadcasted_iota(
            jnp.int32, (N_DELAYS, B, S_PAD), 0) + D_MIN).astype(jnp.float32)
        arrived = S == dvals
        idle = S == 0.0
        newS = fired[None] & idle
        live = (~arrived).astype(jnp.float32)
        S = S * live
        V = V * live

        out_mask = ((col >= N_HIDDEN) & (col < N_NEURONS)).astype(jnp.float32)
        out_ref[...] += Vm * out_mask

        firedf = fired.astype(jnp.float32)
        Vm = Vm - (Vm * firedf + 0.2 * firedf)
        newSf = newS.astype(jnp.float32)
        S = S + (S > 0.0).astype(jnp.float32) * (DT * VMAX) + newSf * (DT * VMAX)
        V = V + newSf * V_exc[None]

        S_ref[...] = S
        Vv_ref[...] = V
        Vm_ref[...] = Vm

        @pl.when(t == T_MAX - 1)
        def _done():
            out_ref[...] = out_ref[...] / jnp.float32(T_MAX)


def kernel(x, W_e, input_W, L_e, src, tgt, key):
    del key
    d_round = jnp.round(L_e)
    is_int = jnp.abs(L_e - d_round) < 0.25
    d_idx = jnp.clip(d_round.astype(jnp.int32) - D_MIN, 0, N_DELAYS - 1)
    w_eff = jnp.where(is_int, W_e, 0.0)
    flat_idx = (d_idx * S_PAD + src) * S_PAD + tgt
    Wd = _build_wd(flat_idx, w_eff)

    input_currents = x.reshape(B, -1) @ input_W
    icur = jnp.pad(input_currents, ((0, 0), (0, S_PAD - N_HIDDEN)))

    s3 = jax.ShapeDtypeStruct((N_DELAYS, B, S_PAD), jnp.float32)
    s2 = jax.ShapeDtypeStruct((B, S_PAD), jnp.float32)
    S0, V0, Vm0, acc0 = pl.pallas_call(
        _warm_kernel,
        out_shape=(s3, s3, s2, s2),
        compiler_params=pltpu.CompilerParams(
            vmem_limit_bytes=100 * 1024 * 1024),
    )(icur)

    full3 = pl.BlockSpec((N_DELAYS, B, S_PAD), lambda t, k: (0, 0, 0))
    full2 = pl.BlockSpec((B, S_PAD), lambda t, k: (0, 0))
    out = pl.pallas_call(
        _snn_kernel,
        grid=(T_MAX - T_WARM, N_DELAYS),
        in_specs=[
            full2,
            pl.BlockSpec((1, S_PAD, S_PAD), lambda t, k: (k, 0, 0)),
            full3, full3, full2, full2,
        ],
        out_specs=full2,
        out_shape=s2,
        scratch_shapes=[
            pltpu.VMEM((N_DELAYS, B, S_PAD), jnp.float32),
            pltpu.VMEM((N_DELAYS, B, S_PAD), jnp.float32),
            pltpu.VMEM((B, S_PAD), jnp.float32),
            pltpu.VMEM((B, S_PAD), jnp.float32),
        ],
        compiler_params=pltpu.CompilerParams(
            dimension_semantics=("arbitrary", "arbitrary"),
            vmem_limit_bytes=100 * 1024 * 1024,
        ),
    )(icur, Wd, S0, V0, Vm0, acc0)
    return out[:, N_HIDDEN:N_NEURONS]
